# baseline (speedup 1.0000x reference)
import functools

import jax
import jax.numpy as jnp
from jax import lax
from jax.experimental import pallas as pl
from jax.experimental.pallas import tpu as pltpu
from jax.experimental.pallas import tpu_sc as plsc

B = 4096
CTX = 20
V = 100000
D = 64
H = 512

NC = 2
NS = 16
NW = NC * NS
N_IDX = B * CTX
IDX_PER_W = N_IDX // NW
CHUNK = 128
NCHUNK = IDX_PER_W // CHUNK

VB = 2048
NV = (V + VB - 1) // VB
BBQ = 512
NQ = B // BBQ


def _gather_body(table_hbm, idx_hbm, out_hbm, idx_v, rows_v, sem):
    wid = lax.axis_index("s") * NC + lax.axis_index("c")
    pltpu.sync_copy(idx_hbm.at[wid], idx_v)
    base = wid * IDX_PER_W

    def chunk(j, carry):
        pltpu.async_copy(table_hbm.at[idx_v.at[j]], rows_v, sem).wait()
        pltpu.sync_copy(rows_v, out_hbm.at[pl.ds(base + j * CHUNK, CHUNK)])
        return carry

    lax.fori_loop(0, NCHUNK, chunk, 0)


def _gather_sc(table, idx_flat):
    mesh = plsc.VectorSubcoreMesh(
        core_axis_name="c", subcore_axis_name="s",
        num_cores=NC, num_subcores=NS)
    return pl.kernel(
        _gather_body,
        out_type=jax.ShapeDtypeStruct((N_IDX, D), jnp.float32),
        mesh=mesh,
        scratch_types=[
            pltpu.VMEM((NCHUNK, CHUNK), jnp.int32),
            pltpu.VMEM((CHUNK, D), jnp.float32),
            pltpu.SemaphoreType.DMA,
        ],
        compiler_params=pltpu.CompilerParams(use_tc_tiling_on_sc=False),
    )(table, idx_flat)


def _h_body(emb_ref, w1_ref, b1_ref, h_ref):
    e = emb_ref[...].astype(jnp.bfloat16)
    w1 = w1_ref[...].astype(jnp.bfloat16)
    acc = lax.dot_general(e, w1, (((1,), (1,)), ((), ())),
                          preferred_element_type=jnp.float32)
    acc = acc + b1_ref[...].reshape(1, H)
    h_ref[...] = jnp.maximum(acc, 0.0).astype(jnp.bfloat16)


def _hidden(embeds, W1, b1):
    return pl.pallas_call(
        _h_body,
        grid=(NQ,),
        in_specs=[
            pl.BlockSpec((BBQ, CTX * D), lambda i: (i, 0)),
            pl.BlockSpec((H, CTX * D), lambda i: (0, 0)),
            pl.BlockSpec((H,), lambda i: (0,)),
        ],
        out_specs=pl.BlockSpec((BBQ, H), lambda i: (i, 0)),
        out_shape=jax.ShapeDtypeStruct((B, H), jnp.bfloat16),
    )(embeds, W1, b1)


def _fused_body(ha_ref, hb_ref, w2_ref, b2_ref, out_ref, m_ref, s_ref,
                lse_ref):
    p = pl.program_id(0)
    v = pl.program_id(1)
    w2 = w2_ref[...]
    b2row = b2_ref[...].reshape(1, VB)

    @pl.when(p < NQ)
    def _():
        rows = pl.ds(p * BBQ, BBQ)

        @pl.when(v == 0)
        def _():
            m_ref[rows, :] = jnp.full((BBQ, 1), -jnp.inf, jnp.float32)
            s_ref[rows, :] = jnp.zeros((BBQ, 1), jnp.float32)

        logits = lax.dot_general(ha_ref[...], w2, (((1,), (1,)), ((), ())),
                                 preferred_element_type=jnp.float32)
        logits = logits + b2row
        col = v * VB + lax.broadcasted_iota(jnp.int32, (1, VB), 1)
        logits = jnp.where(col < V, logits, -jnp.inf)

        m_old = m_ref[rows, :]
        s_old = s_ref[rows, :]
        m_new = jnp.maximum(m_old, jnp.max(logits, axis=1, keepdims=True))
        s_new = s_old * jnp.exp(m_old - m_new) + jnp.sum(
            jnp.exp(logits - m_new), axis=1, keepdims=True)
        m_ref[rows, :] = m_new
        s_ref[rows, :] = s_new

        @pl.when(v == NV - 1)
        def _():
            lse_ref[rows, :] = m_new + jnp.log(s_new)

    @pl.when(p >= 1)
    def _():
        rows = pl.ds((p - 1) * BBQ, BBQ)
        logits2 = lax.dot_general(hb_ref[...], w2, (((1,), (1,)), ((), ())),
                                  preferred_element_type=jnp.float32)
        out_ref[...] = logits2 + b2row - lse_ref[rows, :]


def _fused_softmax(h, W2, b2):
    return pl.pallas_call(
        _fused_body,
        grid=(NQ + 1, NV),
        in_specs=[
            pl.BlockSpec((BBQ, H), lambda p, v: (jnp.minimum(p, NQ - 1), 0)),
            pl.BlockSpec((BBQ, H), lambda p, v: (jnp.maximum(p - 1, 0), 0)),
            pl.BlockSpec((VB, H), lambda p, v: (v, 0)),
            pl.BlockSpec((VB,), lambda p, v: (v,)),
        ],
        out_specs=pl.BlockSpec(
            (BBQ, VB),
            lambda p, v: (jnp.maximum(p - 1, 0), jnp.where(p == 0, 0, v))),
        out_shape=jax.ShapeDtypeStruct((B, V), jnp.float32),
        scratch_shapes=[
            pltpu.VMEM((B, 1), jnp.float32),
            pltpu.VMEM((B, 1), jnp.float32),
            pltpu.VMEM((B, 1), jnp.float32),
        ],
        compiler_params=pltpu.CompilerParams(
            dimension_semantics=("arbitrary", "arbitrary")),
    )(h, h, W2.astype(jnp.bfloat16), b2)


def kernel(inputs, table, W1, b1, W2, b2):
    idx_flat = inputs.astype(jnp.int32).reshape(NW, NCHUNK, CHUNK)
    rows = _gather_sc(table, idx_flat)
    embeds = rows.reshape(B, CTX * D)
    h = _hidden(embeds, W1, b1)
    return _fused_softmax(h, W2, b2)


# speedup vs baseline: 1.0591x; 1.0591x over previous
import functools

import jax
import jax.numpy as jnp
from jax import lax
from jax.experimental import pallas as pl
from jax.experimental.pallas import tpu as pltpu
from jax.experimental.pallas import tpu_sc as plsc

B = 4096
CTX = 20
V = 100000
D = 64
H = 512

NC = 2
NS = 16
NW = NC * NS
N_IDX = B * CTX
IDX_PER_W = N_IDX // NW
CHUNK = 128
NCHUNK = IDX_PER_W // CHUNK

VB = 2048
NV = (V + VB - 1) // VB
BBQ = 1024
NQ = B // BBQ


def _gather_body(table_hbm, idx_hbm, out_hbm, idx_v, rows_v, sem):
    wid = lax.axis_index("s") * NC + lax.axis_index("c")
    pltpu.sync_copy(idx_hbm.at[wid], idx_v)
    base = wid * IDX_PER_W

    def chunk(j, carry):
        pltpu.async_copy(table_hbm.at[idx_v.at[j]], rows_v, sem).wait()
        pltpu.sync_copy(rows_v, out_hbm.at[pl.ds(base + j * CHUNK, CHUNK)])
        return carry

    lax.fori_loop(0, NCHUNK, chunk, 0)


def _gather_sc(table, idx_flat):
    mesh = plsc.VectorSubcoreMesh(
        core_axis_name="c", subcore_axis_name="s",
        num_cores=NC, num_subcores=NS)
    return pl.kernel(
        _gather_body,
        out_type=jax.ShapeDtypeStruct((N_IDX, D), jnp.float32),
        mesh=mesh,
        scratch_types=[
            pltpu.VMEM((NCHUNK, CHUNK), jnp.int32),
            pltpu.VMEM((CHUNK, D), jnp.float32),
            pltpu.SemaphoreType.DMA,
        ],
        compiler_params=pltpu.CompilerParams(use_tc_tiling_on_sc=False),
    )(table, idx_flat)


def _h_body(emb_ref, w1_ref, b1_ref, h_ref):
    e = emb_ref[...].astype(jnp.bfloat16)
    w1 = w1_ref[...].astype(jnp.bfloat16)
    acc = lax.dot_general(e, w1, (((1,), (1,)), ((), ())),
                          preferred_element_type=jnp.float32)
    acc = acc + b1_ref[...].reshape(1, H)
    h_ref[...] = jnp.maximum(acc, 0.0).astype(jnp.bfloat16)


def _hidden(embeds, W1, b1):
    return pl.pallas_call(
        _h_body,
        grid=(NQ,),
        in_specs=[
            pl.BlockSpec((BBQ, CTX * D), lambda i: (i, 0)),
            pl.BlockSpec((H, CTX * D), lambda i: (0, 0)),
            pl.BlockSpec((H,), lambda i: (0,)),
        ],
        out_specs=pl.BlockSpec((BBQ, H), lambda i: (i, 0)),
        out_shape=jax.ShapeDtypeStruct((B, H), jnp.bfloat16),
    )(embeds, W1, b1)


def _fused_body(ha_ref, hb_ref, w2_ref, b2_ref, out_ref, m_ref, s_ref,
                lse_ref):
    p = pl.program_id(0)
    v = pl.program_id(1)
    w2 = w2_ref[...].astype(jnp.bfloat16)
    b2row = b2_ref[...].reshape(1, VB)

    @pl.when(p < NQ)
    def _():
        rows = pl.ds(p * BBQ, BBQ)

        @pl.when(v == 0)
        def _():
            m_ref[rows, :] = jnp.full((BBQ, 1), -jnp.inf, jnp.float32)
            s_ref[rows, :] = jnp.zeros((BBQ, 1), jnp.float32)

        logits = lax.dot_general(ha_ref[...], w2, (((1,), (1,)), ((), ())),
                                 preferred_element_type=jnp.float32)
        logits = logits + b2row
        col = v * VB + lax.broadcasted_iota(jnp.int32, (1, VB), 1)
        logits = jnp.where(col < V, logits, -jnp.inf)

        m_old = m_ref[rows, :]
        s_old = s_ref[rows, :]
        m_new = jnp.maximum(m_old, jnp.max(logits, axis=1, keepdims=True))
        s_new = s_old * jnp.exp(m_old - m_new) + jnp.sum(
            jnp.exp(logits - m_new), axis=1, keepdims=True)
        m_ref[rows, :] = m_new
        s_ref[rows, :] = s_new

        @pl.when(v == NV - 1)
        def _():
            lse_ref[rows, :] = m_new + jnp.log(s_new)

    @pl.when(p >= 1)
    def _():
        rows = pl.ds((p - 1) * BBQ, BBQ)
        logits2 = lax.dot_general(hb_ref[...], w2, (((1,), (1,)), ((), ())),
                                  preferred_element_type=jnp.float32)
        out_ref[...] = logits2 + b2row - lse_ref[rows, :]


def _fused_softmax(h, W2, b2):
    return pl.pallas_call(
        _fused_body,
        grid=(NQ + 1, NV),
        in_specs=[
            pl.BlockSpec((BBQ, H), lambda p, v: (jnp.minimum(p, NQ - 1), 0)),
            pl.BlockSpec((BBQ, H), lambda p, v: (jnp.maximum(p - 1, 0), 0)),
            pl.BlockSpec((VB, H), lambda p, v: (v, 0)),
            pl.BlockSpec((VB,), lambda p, v: (v,)),
        ],
        out_specs=pl.BlockSpec(
            (BBQ, VB),
            lambda p, v: (jnp.maximum(p - 1, 0), jnp.where(p == 0, 0, v))),
        out_shape=jax.ShapeDtypeStruct((B, V), jnp.float32),
        scratch_shapes=[
            pltpu.VMEM((B, 1), jnp.float32),
            pltpu.VMEM((B, 1), jnp.float32),
            pltpu.VMEM((B, 1), jnp.float32),
        ],
        compiler_params=pltpu.CompilerParams(
            dimension_semantics=("arbitrary", "arbitrary")),
    )(h, h, W2, b2)


def kernel(inputs, table, W1, b1, W2, b2):
    idx_flat = inputs.astype(jnp.int32).reshape(NW, NCHUNK, CHUNK)
    rows = _gather_sc(table, idx_flat)
    embeds = rows.reshape(B, CTX * D)
    h = _hidden(embeds, W1, b1)
    return _fused_softmax(h, W2, b2)
